# Initial kernel scaffold; baseline (speedup 1.0000x reference)
import jax
import jax.numpy as jnp
from jax.experimental import pallas as pl


def kernel(inputs, categ_bias, moving_mean, moving_norm):
    raise NotImplementedError("write your pallas kernel here")


# speedup vs baseline: 1.1305x; 1.1305x over previous
import functools

import jax
import jax.numpy as jnp
import numpy as np
from jax import lax
from jax.experimental import pallas as pl
from jax.experimental.pallas import tpu as pltpu
from jax.experimental.pallas import tpu_sc as plsc

_F = 26
_D = 50
_B = 16384
_L = 16
_NC = 2
_NS = 16
_NW = _NC * _NS
_TOTAL = _B * _F
_CHUNK = _TOTAL // _NW
_PERIOD = 208
_INNER = _PERIOD // _L
_OUTER = _CHUNK // _PERIOD
_TAB = _F * _D
_TPAD = ((_TAB + _L - 1) // _L) * _L
_FPAD = 32

_OFFS = np.asarray((np.arange(_PERIOD) % _F) * _D, dtype=np.int32)


def _make_sc_kernel():
    mesh = plsc.VectorSubcoreMesh(core_axis_name="c", subcore_axis_name="s")

    @functools.partial(
        pl.kernel,
        mesh=mesh,
        out_type=jax.ShapeDtypeStruct((_TOTAL,), jnp.float32),
        compiler_params=pltpu.CompilerParams(needs_layout_passes=False),
        scratch_types=[
            pltpu.VMEM((_CHUNK,), jnp.int32),
            pltpu.VMEM((_CHUNK,), jnp.float32),
            pltpu.VMEM((_TPAD,), jnp.float32),
            pltpu.VMEM((_PERIOD,), jnp.int32),
            pltpu.VMEM((_FPAD,), jnp.float32),
            pltpu.VMEM((_FPAD,), jnp.float32),
            pltpu.SemaphoreType.DMA,
        ],
    )
    def sc_kernel(idx_hbm, bias_hbm, mean_hbm, norm_hbm, offs_hbm, out_hbm,
                  idx_v, out_v, tab_v, offs_v, mean_v, norm_v, sem):
        wid = lax.axis_index("s") * _NC + lax.axis_index("c")
        base = wid * _CHUNK

        idx_dma = pltpu.async_copy(idx_hbm.at[pl.ds(base, _CHUNK)], idx_v, sem)
        pltpu.sync_copy(bias_hbm, tab_v.at[pl.ds(0, _TAB)])
        pltpu.sync_copy(mean_hbm, mean_v.at[pl.ds(0, _F)])
        pltpu.sync_copy(norm_hbm, norm_v.at[pl.ds(0, _F)])
        pltpu.sync_copy(offs_hbm, offs_v)

        lane = lax.iota(jnp.int32, _L)
        for t in range(_TPAD // _L):
            sl = pl.ds(t * _L, _L)
            fidx = (lane + t * _L) // _D
            m = plsc.load_gather(mean_v, [fidx])
            nrm = plsc.load_gather(norm_v, [fidx])
            tab_v[sl] = (tab_v[sl] - m) / nrm

        offs_regs = [offs_v[pl.ds(j * _L, _L)] for j in range(_INNER)]

        idx_dma.wait()

        @plsc.parallel_loop(0, _OUTER)
        def gather_loop(o):
            row = o * _PERIOD
            for j in range(_INNER):
                sl = pl.ds(row + j * _L, _L)
                tix = idx_v[sl] + offs_regs[j]
                out_v[sl] = plsc.load_gather(tab_v, [tix])

        pltpu.sync_copy(out_v, out_hbm.at[pl.ds(base, _CHUNK)])

    return sc_kernel


_make_sc_kernel = functools.cache(_make_sc_kernel)


def kernel(inputs, categ_bias, moving_mean, moving_norm):
    idx_flat = jnp.reshape(inputs, (_TOTAL,)).astype(jnp.int32)
    bias_flat = jnp.reshape(categ_bias, (_TAB,))
    mean_flat = jnp.reshape(moving_mean, (_F,))
    norm_flat = jnp.reshape(moving_norm, (_F,))
    offs = jnp.asarray(_OFFS)
    out_flat = _make_sc_kernel()(idx_flat, bias_flat, mean_flat, norm_flat, offs)
    return jnp.reshape(out_flat, (_B, _F))


# speedup vs baseline: 1.6409x; 1.4515x over previous
import functools

import jax
import jax.numpy as jnp
from jax import lax
from jax.experimental import pallas as pl
from jax.experimental.pallas import tpu as pltpu
from jax.experimental.pallas import tpu_sc as plsc

_F = 26
_D = 50
_B = 16384
_L = 16
_NC = 2
_NS = 16
_NW = _NC * _NS
_ROWS = _B // _NW
_NBLK = 4
_RBLK = _ROWS // _NBLK
_TAB = _F * _D
_TPAD = ((_TAB + _L - 1) // _L) * _L
_FPAD = 32
_LO2 = _F - _L


def _make_sc_kernel():
    mesh = plsc.VectorSubcoreMesh(core_axis_name="c", subcore_axis_name="s")

    @functools.partial(
        pl.kernel,
        mesh=mesh,
        out_type=jax.ShapeDtypeStruct((_B, _F), jnp.float32),
        compiler_params=pltpu.CompilerParams(needs_layout_passes=False),
        scratch_types=[
            pltpu.VMEM((2, _RBLK, _F), jnp.int32),
            pltpu.VMEM((2, _RBLK, _F), jnp.float32),
            pltpu.VMEM((_TPAD,), jnp.float32),
            pltpu.VMEM((_FPAD,), jnp.float32),
            pltpu.VMEM((_FPAD,), jnp.float32),
            pltpu.SemaphoreType.DMA,
            pltpu.SemaphoreType.DMA,
            pltpu.SemaphoreType.DMA,
            pltpu.SemaphoreType.DMA,
        ],
    )
    def sc_kernel(idx_hbm, bias_hbm, mean_hbm, norm_hbm, out_hbm,
                  idx_v, out_v, tab_v, mean_v, norm_v,
                  isem0, isem1, osem0, osem1):
        wid = lax.axis_index("s") * _NC + lax.axis_index("c")
        row0 = wid * _ROWS
        isems = [isem0, isem1]
        osems = [osem0, osem1]

        idx_h = [None, None]
        idx_h[0] = pltpu.async_copy(
            idx_hbm.at[pl.ds(row0, _RBLK)], idx_v.at[0], isems[0])
        pltpu.sync_copy(bias_hbm, tab_v.at[pl.ds(0, _TAB)])
        pltpu.sync_copy(mean_hbm, mean_v.at[pl.ds(0, _F)])
        pltpu.sync_copy(norm_hbm, norm_v.at[pl.ds(0, _F)])

        lane = lax.iota(jnp.int32, _L)
        for t in range(_TPAD // _L):
            sl = pl.ds(t * _L, _L)
            fidx = (lane + t * _L) // _D
            m = plsc.load_gather(mean_v, [fidx])
            nrm = plsc.load_gather(norm_v, [fidx])
            tab_v[sl] = (tab_v[sl] - m) / nrm

        off_a = lane * _D
        off_b = (lane + _LO2) * _D

        idx_h[1] = pltpu.async_copy(
            idx_hbm.at[pl.ds(row0 + _RBLK, _RBLK)], idx_v.at[1], isems[1])

        out_h = [None, None]
        for b in range(_NBLK):
            s = b % 2
            idx_h[s].wait()
            if out_h[s] is not None:
                out_h[s].wait()
            ib = idx_v.at[s]
            ob = out_v.at[s]

            @plsc.parallel_loop(0, _RBLK)
            def gather_loop(r):
                tix_a = ib[r, pl.ds(0, _L)] + off_a
                ob[r, pl.ds(0, _L)] = plsc.load_gather(tab_v, [tix_a])
                tix_b = ib[r, pl.ds(_LO2, _L)] + off_b
                ob[r, pl.ds(_LO2, _L)] = plsc.load_gather(tab_v, [tix_b])

            if b + 2 < _NBLK:
                idx_h[s] = pltpu.async_copy(
                    idx_hbm.at[pl.ds(row0 + (b + 2) * _RBLK, _RBLK)],
                    idx_v.at[s], isems[s])
            out_h[s] = pltpu.async_copy(
                ob, out_hbm.at[pl.ds(row0 + b * _RBLK, _RBLK)], osems[s])

        out_h[0].wait()
        out_h[1].wait()

    return sc_kernel


_make_sc_kernel = functools.cache(_make_sc_kernel)


def kernel(inputs, categ_bias, moving_mean, moving_norm):
    bias_flat = jnp.reshape(categ_bias, (_TAB,))
    mean_flat = jnp.reshape(moving_mean, (_F,))
    norm_flat = jnp.reshape(moving_norm, (_F,))
    return _make_sc_kernel()(inputs, bias_flat, mean_flat, norm_flat)


# speedup vs baseline: 2.5001x; 1.5236x over previous
import functools

import jax
import jax.numpy as jnp
from jax import lax
from jax.experimental import pallas as pl
from jax.experimental.pallas import tpu as pltpu
from jax.experimental.pallas import tpu_sc as plsc

_F = 26
_D = 50
_B = 16384
_L = 16
_NC = 2
_NS = 16
_NW = _NC * _NS
_COLS = _B // _NW
_CSTEPS = _COLS // _L
_TAB = _F * _D
_TPAD = ((_TAB + _L - 1) // _L) * _L
_FPAD = 32
_SLAB = _F * _COLS


def _make_sc_kernel():
    mesh = plsc.VectorSubcoreMesh(core_axis_name="c", subcore_axis_name="s")

    @functools.partial(
        pl.kernel,
        mesh=mesh,
        out_type=jax.ShapeDtypeStruct((_F, _B), jnp.float32),
        compiler_params=pltpu.CompilerParams(needs_layout_passes=False),
        scratch_types=[
            pltpu.VMEM((_SLAB,), jnp.int32),
            pltpu.VMEM((_SLAB,), jnp.float32),
            pltpu.VMEM((_TPAD,), jnp.float32),
            pltpu.VMEM((_FPAD,), jnp.float32),
            pltpu.VMEM((_FPAD,), jnp.float32),
            pltpu.SemaphoreType.DMA,
            pltpu.SemaphoreType.DMA,
        ],
    )
    def sc_kernel(idx_hbm, bias_hbm, mean_hbm, norm_hbm, out_hbm,
                  idx_v, out_v, tab_v, mean_v, norm_v, isem, osem):
        wid = lax.axis_index("s") * _NC + lax.axis_index("c")
        col0 = wid * _COLS

        idx_h = [
            pltpu.async_copy(idx_hbm.at[f, pl.ds(col0, _COLS)],
                             idx_v.at[pl.ds(f * _COLS, _COLS)], isem)
            for f in range(_F)
        ]
        pltpu.sync_copy(bias_hbm, tab_v.at[pl.ds(0, _TAB)])
        pltpu.sync_copy(mean_hbm, mean_v.at[pl.ds(0, _F)])
        pltpu.sync_copy(norm_hbm, norm_v.at[pl.ds(0, _F)])

        lane = lax.iota(jnp.int32, _L)
        for t in range(_TPAD // _L):
            sl = pl.ds(t * _L, _L)
            fidx = (lane + t * _L) // _D
            m = plsc.load_gather(mean_v, [fidx])
            nrm = plsc.load_gather(norm_v, [fidx])
            tab_v[sl] = (tab_v[sl] - m) / nrm

        for h in idx_h:
            h.wait()

        @plsc.parallel_loop(0, _CSTEPS)
        def gather_loop(c):
            for f in range(_F):
                sl = pl.ds(f * _COLS + c * _L, _L)
                tix = idx_v[sl] + (f * _D)
                out_v[sl] = plsc.load_gather(tab_v, [tix])

        out_h = [
            pltpu.async_copy(out_v.at[pl.ds(f * _COLS, _COLS)],
                             out_hbm.at[f, pl.ds(col0, _COLS)], osem)
            for f in range(_F)
        ]
        for h in out_h:
            h.wait()

    return sc_kernel


_make_sc_kernel = functools.cache(_make_sc_kernel)


def kernel(inputs, categ_bias, moving_mean, moving_norm):
    idx_t = jnp.transpose(inputs)
    bias_flat = jnp.reshape(categ_bias, (_TAB,))
    mean_flat = jnp.reshape(moving_mean, (_F,))
    norm_flat = jnp.reshape(moving_norm, (_F,))
    out_t = _make_sc_kernel()(idx_t, bias_flat, mean_flat, norm_flat)
    return jnp.transpose(out_t)


# speedup vs baseline: 2.5998x; 1.0399x over previous
import functools

import jax
import jax.numpy as jnp
from jax import lax
from jax.experimental import pallas as pl
from jax.experimental.pallas import tpu as pltpu
from jax.experimental.pallas import tpu_sc as plsc

_F = 26
_D = 50
_B = 16384
_L = 16
_NC = 2
_NS = 16
_NW = _NC * _NS
_COLS = _B // _NW
_CSTEPS = _COLS // _L
_TAB = _F * _D
_TPAD = ((_TAB + _L - 1) // _L) * _L
_DS = 56
_TAB56 = _F * _DS + _L
_FPAD = 32
_SLAB = _F * _COLS


def _make_sc_kernel():
    mesh = plsc.VectorSubcoreMesh(core_axis_name="c", subcore_axis_name="s")

    @functools.partial(
        pl.kernel,
        mesh=mesh,
        out_type=jax.ShapeDtypeStruct((_F, _B), jnp.float32),
        compiler_params=pltpu.CompilerParams(needs_layout_passes=False,
                                             skip_device_barrier=True),
        scratch_types=[
            pltpu.VMEM((_SLAB,), jnp.int32),
            pltpu.VMEM((_SLAB,), jnp.float32),
            pltpu.VMEM((_TPAD,), jnp.float32),
            pltpu.VMEM((_TAB56,), jnp.float32),
            pltpu.VMEM((_FPAD,), jnp.float32),
            pltpu.VMEM((_FPAD,), jnp.float32),
            pltpu.SemaphoreType.DMA,
            pltpu.SemaphoreType.DMA,
        ],
    )
    def sc_kernel(idx_hbm, bias_hbm, mean_hbm, norm_hbm, out_hbm,
                  idx_v, out_v, tab_v, tab56_v, mean_v, norm_v, isem, osem):
        wid = lax.axis_index("s") * _NC + lax.axis_index("c")
        col0 = wid * _COLS

        idx_h = [
            pltpu.async_copy(idx_hbm.at[f, pl.ds(col0, _COLS)],
                             idx_v.at[pl.ds(f * _COLS, _COLS)], isem)
            for f in range(_F)
        ]
        pltpu.sync_copy(bias_hbm, tab_v.at[pl.ds(0, _TAB)])
        pltpu.sync_copy(mean_hbm, mean_v.at[pl.ds(0, _F)])
        pltpu.sync_copy(norm_hbm, norm_v.at[pl.ds(0, _F)])

        lane = lax.iota(jnp.int32, _L)

        def prescale(t, carry):
            flat = lane + t * _L
            fidx = flat // _D
            m = plsc.load_gather(mean_v, [fidx])
            nrm = plsc.load_gather(norm_v, [fidx])
            val = (tab_v[pl.ds(t * _L, _L)] - m) / nrm
            plsc.store_scatter(tab56_v, [flat + fidx * (_DS - _D)], val)
            return carry

        lax.fori_loop(0, _TPAD // _L, prescale, 0)

        for h in idx_h:
            h.wait()

        tab_f = [tab56_v.at[pl.ds(f * _DS, _DS)] for f in range(_F)]

        @plsc.parallel_loop(0, _CSTEPS)
        def gather_loop(c):
            for f in range(_F):
                sl = pl.ds(f * _COLS + c * _L, _L)
                out_v[sl] = plsc.load_gather(tab_f[f], [idx_v[sl]])

        out_h = [
            pltpu.async_copy(out_v.at[pl.ds(f * _COLS, _COLS)],
                             out_hbm.at[f, pl.ds(col0, _COLS)], osem)
            for f in range(_F)
        ]
        for h in out_h:
            h.wait()

    return sc_kernel


_make_sc_kernel = functools.cache(_make_sc_kernel)


def kernel(inputs, categ_bias, moving_mean, moving_norm):
    idx_t = jnp.transpose(inputs)
    bias_flat = jnp.reshape(categ_bias, (_TAB,))
    mean_flat = jnp.reshape(moving_mean, (_F,))
    norm_flat = jnp.reshape(moving_norm, (_F,))
    out_t = _make_sc_kernel()(idx_t, bias_flat, mean_flat, norm_flat)
    return jnp.transpose(out_t)


# speedup vs baseline: 2.6389x; 1.0150x over previous
import functools

import jax
import jax.numpy as jnp
from jax import lax
from jax.experimental import pallas as pl
from jax.experimental.pallas import tpu as pltpu
from jax.experimental.pallas import tpu_sc as plsc

_F = 26
_D = 50
_B = 16384
_L = 16
_NC = 2
_NS = 16
_NW = _NC * _NS
_COLS = _B // _NW
_CSTEPS = _COLS // _L
_TAB = _F * _D
_TPAD = ((_TAB + _L - 1) // _L) * _L
_DS = 56
_TAB56 = _F * _DS + _L
_FPAD = 32
_SLAB = _F * _COLS


def _make_sc_kernel():
    mesh = plsc.VectorSubcoreMesh(core_axis_name="c", subcore_axis_name="s")

    @functools.partial(
        pl.kernel,
        mesh=mesh,
        out_type=jax.ShapeDtypeStruct((_F, _B), jnp.float32),
        compiler_params=pltpu.CompilerParams(needs_layout_passes=False,
                                             skip_device_barrier=True),
        scratch_types=[
            pltpu.VMEM((_SLAB,), jnp.int32),
            pltpu.VMEM((_SLAB,), jnp.float32),
            pltpu.VMEM((_TPAD,), jnp.float32),
            pltpu.VMEM((_TAB56,), jnp.float32),
            pltpu.VMEM((_FPAD,), jnp.float32),
            pltpu.VMEM((_FPAD,), jnp.float32),
            pltpu.SemaphoreType.DMA,
            pltpu.SemaphoreType.DMA,
            pltpu.SemaphoreType.DMA,
        ],
    )
    def sc_kernel(idx_hbm, bias_hbm, mean_hbm, norm_hbm, out_hbm,
                  idx_v, out_v, tab_v, tab56_v, mean_v, norm_v,
                  isem, osem, bsem):
        wid = lax.axis_index("s") * _NC + lax.axis_index("c")
        col0 = wid * _COLS

        idx_h = [
            pltpu.async_copy(idx_hbm.at[f, pl.ds(col0, _COLS)],
                             idx_v.at[pl.ds(f * _COLS, _COLS)], isem)
            for f in range(_F)
        ]
        bias_h = pltpu.async_copy(bias_hbm, tab_v.at[pl.ds(0, _TAB)], bsem)
        pltpu.sync_copy(mean_hbm, mean_v.at[pl.ds(0, _F)])
        pltpu.sync_copy(norm_hbm, norm_v.at[pl.ds(0, _F)])
        bias_h.wait()

        lane = lax.iota(jnp.int32, _L)

        def prescale(t, carry):
            flat = lane + t * _L
            fidx = flat // _D
            m = plsc.load_gather(mean_v, [fidx])
            nrm = plsc.load_gather(norm_v, [fidx])
            val = (tab_v[pl.ds(t * _L, _L)] - m) / nrm
            plsc.store_scatter(tab56_v, [flat + fidx * (_DS - _D)], val)
            return carry

        lax.fori_loop(0, _TPAD // _L, prescale, 0)

        for h in idx_h:
            h.wait()

        tab_f = [tab56_v.at[pl.ds(f * _DS, _DS)] for f in range(_F)]

        @plsc.parallel_loop(0, _CSTEPS, unroll=2)
        def gather_loop(c):
            for f in range(_F):
                sl = pl.ds(f * _COLS + c * _L, _L)
                out_v[sl] = plsc.load_gather(tab_f[f], [idx_v[sl]])

        out_h = [
            pltpu.async_copy(out_v.at[pl.ds(f * _COLS, _COLS)],
                             out_hbm.at[f, pl.ds(col0, _COLS)], osem)
            for f in range(_F)
        ]
        for h in out_h:
            h.wait()

    return sc_kernel


_make_sc_kernel = functools.cache(_make_sc_kernel)


def kernel(inputs, categ_bias, moving_mean, moving_norm):
    idx_t = jnp.transpose(inputs)
    bias_flat = jnp.reshape(categ_bias, (_TAB,))
    mean_flat = jnp.reshape(moving_mean, (_F,))
    norm_flat = jnp.reshape(moving_norm, (_F,))
    out_t = _make_sc_kernel()(idx_t, bias_flat, mean_flat, norm_flat)
    return jnp.transpose(out_t)
